# baseline (speedup 1.0000x reference)
import functools

import jax
import jax.numpy as jnp
from jax import lax
from jax.experimental import pallas as pl
from jax.experimental.pallas import tpu as pltpu
from jax.experimental.pallas import tpu_sc as plsc

EMBED = 64
SCALE = 8.0
NC, NS = 2, 16
CHUNK = 128


def _rsqrt_newton(x):
    i = plsc.bitcast(x, jnp.int32)
    i = jnp.int32(0x5F3759DF) - lax.shift_right_logical(i, 1)
    y = plsc.bitcast(i, jnp.float32)
    for _ in range(3):
        y = y * (jnp.float32(1.5) - jnp.float32(0.5) * x * y * y)
    return y


def _normalize_chunk(rows_v):
    lanes = lax.iota(jnp.int32, 16)
    perm = [lanes ^ (1 << k) for k in range(4)]

    @pl.loop(0, CHUNK)
    def _row(r):
        vs = [rows_v[r, pl.ds(k * 16, 16)] for k in range(EMBED // 16)]
        ssv = vs[0] * vs[0]
        for v in vs[1:]:
            ssv = ssv + v * v
        for p in perm:
            ssv = ssv + jnp.take(ssv, p)
        x = jnp.maximum(ssv, jnp.float32(1e-30))
        norm = x * _rsqrt_newton(x)
        factor = jnp.float32(SCALE) / jnp.maximum(norm, jnp.float32(1e-12))
        for k, v in enumerate(vs):
            rows_v[r, pl.ds(k * 16, 16)] = v * factor


def _sc_embed(idx2d, weight, *, interpret=False):
    R, C = idx2d.shape
    NW = NC * NS
    rpw = R // NW
    B = R * C

    mesh = plsc.VectorSubcoreMesh(core_axis_name="c", subcore_axis_name="s")

    @functools.partial(
        pl.kernel,
        out_type=jax.ShapeDtypeStruct((B, EMBED), jnp.float32),
        mesh=mesh,
        scratch_types=[
            pltpu.VMEM((rpw, C), jnp.int32),
            pltpu.VMEM((C, EMBED), jnp.float32),
            pltpu.SemaphoreType.DMA,
        ],
        compiler_params=pltpu.CompilerParams(
            needs_layout_passes=False, use_tc_tiling_on_sc=False
        ),
        interpret=interpret,
    )
    def k(w_hbm, idx_hbm, out_hbm, idx_v, rows_v, sem):
        wid = lax.axis_index("s") * NC + lax.axis_index("c")
        pltpu.sync_copy(idx_hbm.at[pl.ds(wid * rpw, rpw)], idx_v)

        @pl.loop(0, rpw)
        def _chunk(j):
            pltpu.async_copy(w_hbm.at[idx_v.at[j]], rows_v, sem).wait()
            _normalize_chunk(rows_v)
            pltpu.sync_copy(rows_v, out_hbm.at[pl.ds((wid * rpw + j) * C, C)])

    return k(weight, idx2d)


def kernel(x, weight):
    s0, s1 = x.shape
    b = s0 * s1
    idx2d = x.reshape(b // CHUNK, CHUNK).astype(jnp.int32)
    out = _sc_embed(idx2d, weight)
    return out.reshape(s0, s1, EMBED)


# speedup vs baseline: 1.0891x; 1.0891x over previous
import functools

import jax
import jax.numpy as jnp
from jax import lax
from jax.experimental import pallas as pl
from jax.experimental.pallas import tpu as pltpu
from jax.experimental.pallas import tpu_sc as plsc

EMBED = 64
SCALE = 8.0
NC, NS = 2, 16
CHUNK = 128


def _rsqrt_newton(x):
    i = plsc.bitcast(x, jnp.int32)
    i = jnp.int32(0x5F3759DF) - lax.shift_right_logical(i, 1)
    y = plsc.bitcast(i, jnp.float32)
    for _ in range(3):
        y = y * (jnp.float32(1.5) - jnp.float32(0.5) * x * y * y)
    return y


def _normalize_chunk(rows_v):
    lanes = lax.iota(jnp.int32, 16)
    perm = [lanes ^ (1 << k) for k in range(4)]

    @pl.loop(0, CHUNK, unroll=8)
    def _row(r):
        vs = [rows_v[r, pl.ds(k * 16, 16)] for k in range(EMBED // 16)]
        ssv = vs[0] * vs[0]
        for v in vs[1:]:
            ssv = ssv + v * v
        for p in perm:
            ssv = ssv + jnp.take(ssv, p)
        x = jnp.maximum(ssv, jnp.float32(1e-30))
        norm = x * _rsqrt_newton(x)
        factor = jnp.float32(SCALE) / jnp.maximum(norm, jnp.float32(1e-12))
        for k, v in enumerate(vs):
            rows_v[r, pl.ds(k * 16, 16)] = v * factor


def _sc_embed(idx2d, weight, *, interpret=False):
    R, C = idx2d.shape
    NW = NC * NS
    rpw = R // NW
    B = R * C

    mesh = plsc.VectorSubcoreMesh(core_axis_name="c", subcore_axis_name="s")

    @functools.partial(
        pl.kernel,
        out_type=jax.ShapeDtypeStruct((B, EMBED), jnp.float32),
        mesh=mesh,
        scratch_types=[
            pltpu.VMEM((rpw, C), jnp.int32),
            pltpu.VMEM((C, EMBED), jnp.float32),
            pltpu.VMEM((C, EMBED), jnp.float32),
            pltpu.SemaphoreType.DMA,
            pltpu.SemaphoreType.DMA,
            pltpu.SemaphoreType.DMA,
            pltpu.SemaphoreType.DMA,
        ],
        compiler_params=pltpu.CompilerParams(
            needs_layout_passes=False, use_tc_tiling_on_sc=False
        ),
        interpret=interpret,
    )
    def k(w_hbm, idx_hbm, out_hbm, idx_v, rows_a, rows_b, sia, sib, soa, sob):
        wid = lax.axis_index("s") * NC + lax.axis_index("c")
        pltpu.sync_copy(idx_hbm.at[pl.ds(wid * rpw, rpw)], idx_v)
        base = wid * rpw

        def gather(j, buf, sem):
            pltpu.async_copy(w_hbm.at[idx_v.at[j]], buf, sem)

        def put(j, buf, sem):
            pltpu.async_copy(buf, out_hbm.at[pl.ds((base + j) * C, C)], sem)

        def wait(j, buf, sem):
            pltpu.make_async_copy(buf, out_hbm.at[pl.ds((base + j) * C, C)], sem).wait()

        def wait_in(j, buf, sem):
            pltpu.make_async_copy(w_hbm.at[idx_v.at[j]], buf, sem).wait()

        gather(0, rows_a, sia)

        @pl.loop(0, rpw // 2)
        def _it(g):
            j0 = 2 * g

            @pl.when(g > 0)
            def _():
                wait(j0 - 1, rows_b, sob)

            gather(j0 + 1, rows_b, sib)
            wait_in(j0, rows_a, sia)
            _normalize_chunk(rows_a)
            put(j0, rows_a, soa)
            wait(j0, rows_a, soa)

            @pl.when(j0 + 2 < rpw)
            def _():
                gather(j0 + 2, rows_a, sia)

            wait_in(j0 + 1, rows_b, sib)
            _normalize_chunk(rows_b)
            put(j0 + 1, rows_b, sob)

        wait(rpw - 1, rows_b, sob)

    return k(weight, idx2d)


def kernel(x, weight):
    s0, s1 = x.shape
    b = s0 * s1
    idx2d = x.reshape(b // CHUNK, CHUNK).astype(jnp.int32)
    out = _sc_embed(idx2d, weight)
    return out.reshape(s0, s1, EMBED)


# speedup vs baseline: 1.7417x; 1.5992x over previous
import functools

import jax
import jax.numpy as jnp
from jax import lax
from jax.experimental import pallas as pl
from jax.experimental.pallas import tpu as pltpu
from jax.experimental.pallas import tpu_sc as plsc

EMBED = 64
SCALE = 8.0
NC, NS = 2, 16
CHUNK = 128


def _rsqrt_newton(x):
    i = plsc.bitcast(x, jnp.int32)
    i = jnp.int32(0x5F3759DF) - lax.shift_right_logical(i, 1)
    y = plsc.bitcast(i, jnp.float32)
    for _ in range(3):
        y = y * (jnp.float32(1.5) - jnp.float32(0.5) * x * y * y)
    return y


def _normalize_chunk(rows_v):
    lanes = lax.iota(jnp.int32, 16)
    perm = [lanes ^ (1 << k) for k in range(4)]

    @plsc.parallel_loop(0, CHUNK, unroll=8)
    def _row(r):
        vs = [rows_v[r, pl.ds(k * 16, 16)] for k in range(EMBED // 16)]
        ssv = vs[0] * vs[0]
        for v in vs[1:]:
            ssv = ssv + v * v
        for p in perm:
            ssv = ssv + jnp.take(ssv, p)
        x = jnp.maximum(ssv, jnp.float32(1e-30))
        norm = x * _rsqrt_newton(x)
        factor = jnp.float32(SCALE) / jnp.maximum(norm, jnp.float32(1e-12))
        for k, v in enumerate(vs):
            rows_v[r, pl.ds(k * 16, 16)] = v * factor


def _sc_embed(idx2d, weight, *, interpret=False):
    R, C = idx2d.shape
    NW = NC * NS
    rpw = R // NW
    B = R * C

    mesh = plsc.VectorSubcoreMesh(core_axis_name="c", subcore_axis_name="s")

    @functools.partial(
        pl.kernel,
        out_type=jax.ShapeDtypeStruct((B, EMBED), jnp.float32),
        mesh=mesh,
        scratch_types=[
            pltpu.VMEM((rpw, C), jnp.int32),
            pltpu.VMEM((C, EMBED), jnp.float32),
            pltpu.VMEM((C, EMBED), jnp.float32),
            pltpu.SemaphoreType.DMA,
            pltpu.SemaphoreType.DMA,
            pltpu.SemaphoreType.DMA,
            pltpu.SemaphoreType.DMA,
        ],
        compiler_params=pltpu.CompilerParams(
            needs_layout_passes=False, use_tc_tiling_on_sc=False
        ),
        interpret=interpret,
    )
    def k(w_hbm, idx_hbm, out_hbm, idx_v, rows_a, rows_b, sia, sib, soa, sob):
        wid = lax.axis_index("s") * NC + lax.axis_index("c")
        pltpu.sync_copy(idx_hbm.at[pl.ds(wid * rpw, rpw)], idx_v)
        base = wid * rpw

        def gather(j, buf, sem):
            pltpu.async_copy(w_hbm.at[idx_v.at[j]], buf, sem)

        def put(j, buf, sem):
            pltpu.async_copy(buf, out_hbm.at[pl.ds((base + j) * C, C)], sem)

        def wait(j, buf, sem):
            pltpu.make_async_copy(buf, out_hbm.at[pl.ds((base + j) * C, C)], sem).wait()

        def wait_in(j, buf, sem):
            pltpu.make_async_copy(w_hbm.at[idx_v.at[j]], buf, sem).wait()

        gather(0, rows_a, sia)

        @pl.loop(0, rpw // 2)
        def _it(g):
            j0 = 2 * g

            @pl.when(g > 0)
            def _():
                wait(j0 - 1, rows_b, sob)

            gather(j0 + 1, rows_b, sib)
            wait_in(j0, rows_a, sia)
            _normalize_chunk(rows_a)
            put(j0, rows_a, soa)
            wait(j0, rows_a, soa)

            @pl.when(j0 + 2 < rpw)
            def _():
                gather(j0 + 2, rows_a, sia)

            wait_in(j0 + 1, rows_b, sib)
            _normalize_chunk(rows_b)
            put(j0 + 1, rows_b, sob)

        wait(rpw - 1, rows_b, sob)

    return k(weight, idx2d)


def kernel(x, weight):
    s0, s1 = x.shape
    b = s0 * s1
    idx2d = x.reshape(b // CHUNK, CHUNK).astype(jnp.int32)
    out = _sc_embed(idx2d, weight)
    return out.reshape(s0, s1, EMBED)


# speedup vs baseline: 1.7463x; 1.0026x over previous
import functools

import jax
import jax.numpy as jnp
from jax import lax
from jax.experimental import pallas as pl
from jax.experimental.pallas import tpu as pltpu
from jax.experimental.pallas import tpu_sc as plsc

EMBED = 64
SCALE = 8.0
NC, NS = 2, 16
CHUNK = 128


def _rsqrt_newton(x):
    i = plsc.bitcast(x, jnp.int32)
    i = jnp.int32(0x5F3759DF) - lax.shift_right_logical(i, 1)
    y = plsc.bitcast(i, jnp.float32)
    for _ in range(3):
        y = y * (jnp.float32(1.5) - jnp.float32(0.5) * x * y * y)
    return y


def _normalize_chunk(rows_v):
    lanes = lax.iota(jnp.int32, 16)
    perm = [lanes ^ (1 << k) for k in range(4)]

    @plsc.parallel_loop(0, CHUNK, unroll=8)
    def _row(r):
        vs = [rows_v[r, pl.ds(k * 16, 16)] for k in range(EMBED // 16)]
        ssv = vs[0] * vs[0]
        for v in vs[1:]:
            ssv = ssv + v * v
        for p in perm:
            ssv = ssv + jnp.take(ssv, p)
        x = jnp.maximum(ssv, jnp.float32(1e-30))
        norm = x * _rsqrt_newton(x)
        factor = jnp.float32(SCALE) / jnp.maximum(norm, jnp.float32(1e-12))
        for k, v in enumerate(vs):
            rows_v[r, pl.ds(k * 16, 16)] = v * factor


def _sc_embed(idx2d, weight, *, interpret=False):
    R, C = idx2d.shape
    NW = NC * NS
    rpw = R // NW
    B = R * C

    mesh = plsc.VectorSubcoreMesh(core_axis_name="c", subcore_axis_name="s")

    @functools.partial(
        pl.kernel,
        out_type=jax.ShapeDtypeStruct((B, EMBED), jnp.float32),
        mesh=mesh,
        scratch_types=[
            pltpu.VMEM((rpw, C), jnp.int32),
            pltpu.VMEM((C, EMBED), jnp.float32),
            pltpu.VMEM((C, EMBED), jnp.float32),
            pltpu.SemaphoreType.DMA,
            pltpu.SemaphoreType.DMA,
            pltpu.SemaphoreType.DMA,
            pltpu.SemaphoreType.DMA,
        ],
        compiler_params=pltpu.CompilerParams(
            needs_layout_passes=False,
            use_tc_tiling_on_sc=False,
            skip_device_barrier=True,
        ),
        interpret=interpret,
    )
    def k(w_hbm, idx_hbm, out_hbm, idx_v, rows_a, rows_b, sia, sib, soa, sob):
        wid = lax.axis_index("s") * NC + lax.axis_index("c")
        pltpu.sync_copy(idx_hbm.at[pl.ds(wid * rpw, rpw)], idx_v)
        base = wid * rpw

        def gather(j, buf, sem):
            pltpu.async_copy(w_hbm.at[idx_v.at[j]], buf, sem)

        def put(j, buf, sem):
            pltpu.async_copy(buf, out_hbm.at[pl.ds((base + j) * C, C)], sem)

        def wait(j, buf, sem):
            pltpu.make_async_copy(buf, out_hbm.at[pl.ds((base + j) * C, C)], sem).wait()

        def wait_in(j, buf, sem):
            pltpu.make_async_copy(w_hbm.at[idx_v.at[j]], buf, sem).wait()

        gather(0, rows_a, sia)

        @pl.loop(0, rpw // 2)
        def _it(g):
            j0 = 2 * g

            @pl.when(g > 0)
            def _():
                wait(j0 - 1, rows_b, sob)

            gather(j0 + 1, rows_b, sib)
            wait_in(j0, rows_a, sia)
            _normalize_chunk(rows_a)
            put(j0, rows_a, soa)
            wait(j0, rows_a, soa)

            @pl.when(j0 + 2 < rpw)
            def _():
                gather(j0 + 2, rows_a, sia)

            wait_in(j0 + 1, rows_b, sib)
            _normalize_chunk(rows_b)
            put(j0 + 1, rows_b, sob)

        wait(rpw - 1, rows_b, sob)

    return k(weight, idx2d)


def kernel(x, weight):
    s0, s1 = x.shape
    b = s0 * s1
    idx2d = x.reshape(b // CHUNK, CHUNK).astype(jnp.int32)
    out = _sc_embed(idx2d, weight)
    return out.reshape(s0, s1, EMBED)


# speedup vs baseline: 1.7514x; 1.0029x over previous
import functools

import jax
import jax.numpy as jnp
from jax import lax
from jax.experimental import pallas as pl
from jax.experimental.pallas import tpu as pltpu
from jax.experimental.pallas import tpu_sc as plsc

EMBED = 64
SCALE = 8.0
NC, NS = 2, 16
G = 100


def _rsqrt_newton(x):
    i = plsc.bitcast(x, jnp.int32)
    i = jnp.int32(0x5F3759DF) - lax.shift_right_logical(i, 1)
    y = plsc.bitcast(i, jnp.float32)
    for _ in range(3):
        y = y * (jnp.float32(1.5) - jnp.float32(0.5) * x * y * y)
    return y


def _normalize(rows_v, n):
    lanes = lax.iota(jnp.int32, 16)
    perm = [lanes ^ (1 << k) for k in range(4)]

    @plsc.parallel_loop(0, n, unroll=8)
    def _row(r):
        vs = [rows_v[r, pl.ds(k * 16, 16)] for k in range(EMBED // 16)]
        ssv = vs[0] * vs[0]
        for v in vs[1:]:
            ssv = ssv + v * v
        for p in perm:
            ssv = ssv + jnp.take(ssv, p)
        x = jnp.maximum(ssv, jnp.float32(1e-30))
        norm = x * _rsqrt_newton(x)
        factor = jnp.float32(SCALE) / jnp.maximum(norm, jnp.float32(1e-12))
        for k, v in enumerate(vs):
            rows_v[r, pl.ds(k * 16, 16)] = v * factor


def _sc_embed(idx2d, weight, nb, t):
    NW = NC * NS
    bpw = nb // NW
    rpb = 2 * bpw

    mesh = plsc.VectorSubcoreMesh(core_axis_name="c", subcore_axis_name="s")

    @functools.partial(
        pl.kernel,
        out_type=jax.ShapeDtypeStruct((nb, t, EMBED), jnp.float32),
        mesh=mesh,
        scratch_types=[
            pltpu.VMEM((rpb, G), jnp.int32),
            pltpu.VMEM((t, EMBED), jnp.float32),
            pltpu.VMEM((t, EMBED), jnp.float32),
            pltpu.SemaphoreType.DMA,
            pltpu.SemaphoreType.DMA,
            pltpu.SemaphoreType.DMA,
            pltpu.SemaphoreType.DMA,
        ],
        compiler_params=pltpu.CompilerParams(
            needs_layout_passes=False,
            use_tc_tiling_on_sc=False,
        ),
    )
    def k(w_hbm, idx_hbm, out_hbm, idx_v, rows_a, rows_b, sia, sib, soa, sob):
        wid = lax.axis_index("s") * NC + lax.axis_index("c")
        pltpu.sync_copy(idx_hbm.at[pl.ds(wid * rpb, rpb)], idx_v)
        base = wid * bpw

        def gather(j, buf, sem):
            pltpu.async_copy(w_hbm.at[idx_v.at[2 * j]], buf.at[pl.ds(0, G)], sem)
            pltpu.async_copy(w_hbm.at[idx_v.at[2 * j + 1]], buf.at[pl.ds(G, G)], sem)

        def wait_in(j, buf, sem):
            pltpu.make_async_copy(w_hbm.at[idx_v.at[2 * j]], buf.at[pl.ds(0, G)], sem).wait()
            pltpu.make_async_copy(w_hbm.at[idx_v.at[2 * j + 1]], buf.at[pl.ds(G, G)], sem).wait()

        def put(j, buf, sem):
            pltpu.async_copy(buf, out_hbm.at[base + j], sem)

        def wait_out(j, buf, sem):
            pltpu.make_async_copy(buf, out_hbm.at[base + j], sem).wait()

        gather(0, rows_a, sia)

        @pl.loop(0, bpw // 2)
        def _it(g):
            j0 = 2 * g

            @pl.when(g > 0)
            def _():
                wait_out(j0 - 1, rows_b, sob)

            gather(j0 + 1, rows_b, sib)
            wait_in(j0, rows_a, sia)
            _normalize(rows_a, t)
            put(j0, rows_a, soa)
            wait_out(j0, rows_a, soa)

            @pl.when(j0 + 2 < bpw)
            def _():
                gather(j0 + 2, rows_a, sia)

            wait_in(j0 + 1, rows_b, sib)
            _normalize(rows_b, t)
            put(j0 + 1, rows_b, sob)

        wait_out(bpw - 1, rows_b, sob)

    return k(weight, idx2d)


def kernel(x, weight):
    nb, t = x.shape
    idx2d = x.reshape(nb * 2, G).astype(jnp.int32)
    return _sc_embed(idx2d, weight, nb, t)


# speedup vs baseline: 2.1980x; 1.2550x over previous
import functools

import jax
import jax.numpy as jnp
from jax import lax
from jax.experimental import pallas as pl
from jax.experimental.pallas import tpu as pltpu
from jax.experimental.pallas import tpu_sc as plsc

EMBED = 64
SCALE = 8.0
NC, NS = 2, 16
CHUNK = 128


def _rsqrt_newton(x):
    i = plsc.bitcast(x, jnp.int32)
    i = jnp.int32(0x5F3759DF) - lax.shift_right_logical(i, 1)
    y = plsc.bitcast(i, jnp.float32)
    for _ in range(3):
        y = y * (jnp.float32(1.5) - jnp.float32(0.5) * x * y * y)
    return y


def _normalize(big_v, offs_v, out_v):
    lanes = lax.iota(jnp.int32, 16)
    perm = [lanes ^ (1 << k) for k in range(4)]

    @plsc.parallel_loop(0, CHUNK, unroll=8)
    def _row(r):
        o = offs_v[pl.ds(r, 16)][0]
        vs = [big_v[r, pl.ds(o + k * 16, 16)] for k in range(EMBED // 16)]
        ssv = vs[0] * vs[0]
        for v in vs[1:]:
            ssv = ssv + v * v
        for p in perm:
            ssv = ssv + jnp.take(ssv, p)
        x = jnp.maximum(ssv, jnp.float32(1e-30))
        norm = x * _rsqrt_newton(x)
        factor = jnp.float32(SCALE) / jnp.maximum(norm, jnp.float32(1e-12))
        for k, v in enumerate(vs):
            out_v[r, pl.ds(k * 16, 16)] = v * factor


def _sc_embed(idx2d, w2):
    R, C = idx2d.shape
    NW = NC * NS
    rpw = R // NW
    B = R * C

    mesh = plsc.VectorSubcoreMesh(core_axis_name="c", subcore_axis_name="s")

    @functools.partial(
        pl.kernel,
        out_type=jax.ShapeDtypeStruct((B, EMBED), jnp.float32),
        mesh=mesh,
        scratch_types=[
            pltpu.VMEM((rpw, C), jnp.int32),
            pltpu.VMEM((C, 2 * EMBED), jnp.float32),
            pltpu.VMEM((C, 2 * EMBED), jnp.float32),
            pltpu.VMEM((C, EMBED), jnp.float32),
            pltpu.VMEM((C, EMBED), jnp.float32),
            pltpu.VMEM((C,), jnp.int32),
            pltpu.VMEM((C,), jnp.int32),
            pltpu.VMEM((C + 16,), jnp.int32),
            pltpu.VMEM((C + 16,), jnp.int32),
            pltpu.SemaphoreType.DMA,
            pltpu.SemaphoreType.DMA,
            pltpu.SemaphoreType.DMA,
            pltpu.SemaphoreType.DMA,
        ],
        compiler_params=pltpu.CompilerParams(
            needs_layout_passes=False,
            use_tc_tiling_on_sc=True,
        ),
    )
    def k(w_hbm, idx_hbm, out_hbm, idx_v, big_a, big_b, out_a, out_b,
          ids_a, ids_b, off_a, off_b, sia, sib, soa, sob):
        wid = lax.axis_index("s") * NC + lax.axis_index("c")
        pltpu.sync_copy(idx_hbm.at[pl.ds(wid * rpw, rpw)], idx_v)
        base = wid * rpw

        def prep(j, ids, off):
            for kk in range(C // 16):
                v = idx_v[j, pl.ds(kk * 16, 16)]
                ids[pl.ds(kk * 16, 16)] = lax.shift_right_logical(v, 1)
                off[pl.ds(kk * 16, 16)] = lax.shift_left(v & 1, 6)

        def gather(buf, ids, sem):
            pltpu.async_copy(w_hbm.at[ids], buf, sem)

        def wait_in(buf, ids, sem):
            pltpu.make_async_copy(w_hbm.at[ids], buf, sem).wait()

        def put(j, buf, sem):
            pltpu.async_copy(buf, out_hbm.at[pl.ds((base + j) * C, C)], sem)

        def wait_out(j, buf, sem):
            pltpu.make_async_copy(
                buf, out_hbm.at[pl.ds((base + j) * C, C)], sem).wait()

        prep(0, ids_a, off_a)
        gather(big_a, ids_a, sia)

        @pl.loop(0, rpw // 2)
        def _it(g):
            j0 = 2 * g

            prep(j0 + 1, ids_b, off_b)
            gather(big_b, ids_b, sib)

            wait_in(big_a, ids_a, sia)

            @pl.when(g > 0)
            def _():
                wait_out(j0 - 2, out_a, soa)

            _normalize(big_a, off_a, out_a)
            put(j0, out_a, soa)

            @pl.when(j0 + 2 < rpw)
            def _():
                prep(j0 + 2, ids_a, off_a)
                gather(big_a, ids_a, sia)

            wait_in(big_b, ids_b, sib)

            @pl.when(g > 0)
            def _():
                wait_out(j0 - 1, out_b, sob)

            _normalize(big_b, off_b, out_b)
            put(j0 + 1, out_b, sob)

        wait_out(rpw - 2, out_a, soa)
        wait_out(rpw - 1, out_b, sob)

    return k(w2, idx2d)


def kernel(x, weight):
    nb, t = x.shape
    b = nb * t
    idx2d = x.reshape(b // CHUNK, CHUNK).astype(jnp.int32)
    w2 = weight.reshape(weight.shape[0] // 2, 2 * EMBED)
    out = _sc_embed(idx2d, w2)
    return out.reshape(nb, t, EMBED)
